# Initial kernel scaffold; baseline (speedup 1.0000x reference)
import jax
import jax.numpy as jnp
from jax.experimental import pallas as pl


def kernel(x, edge_index, edge_attr, W1, root1, b1, g1, be1, rm1, rv1, W2, root2, b2, g2, be2, rm2, rv2, W3, root3, b3, g3, be3, rm3, rv3, W4, root4, b4):
    raise NotImplementedError("write your pallas kernel here")


# speedup vs baseline: 1.3633x; 1.3633x over previous
import functools
import jax
import jax.numpy as jnp
from jax import lax
from jax.experimental import pallas as pl
from jax.experimental.pallas import tpu as pltpu

KS = 25
DIM = 3
S = 8
KC = KS ** DIM
KB = 8
GKEYS = 15632
G = GKEYS // KB
R = 256
DPAD = 128
EPS = 1e-5
N_NODES = 10000
NSEG = 10240


def _basis_kidx(edge_attr):
    u = jnp.clip(edge_attr, 0.0, 1.0)
    v = u * (KS - 1)
    lo = jnp.floor(v)
    frac = v - lo
    lo_i = lo.astype(jnp.int32)
    bits = jnp.array([[(s >> d) & 1 for d in range(DIM)] for s in range(S)],
                     jnp.int32)
    fd = frac[:, None, :]
    w = jnp.prod(jnp.where(bits[None] == 1, fd, 1.0 - fd), axis=-1)
    strides = jnp.array([1, KS, KS * KS], jnp.int32)
    idx = jnp.sum(jnp.clip(lo_i[:, None, :] + bits[None], 0, KS - 1)
                  * strides, axis=-1)
    return w, idx.astype(jnp.int32)


def _grouped_kernel(ci, co, off_ref, y_hbm, k_hbm, w_hbm, W_ref, z_hbm,
                    y_v, k_v, w_v, z_v, s0_, s1_, s2_, s3_):
    g = pl.program_id(0)
    start = off_ref[g * KB]
    end = off_ref[(g + 1) * KB]
    nch = (end - start + (R - 1)) // R

    def body(i, carry):
        s0 = start + i * R
        c1 = pltpu.make_async_copy(y_hbm.at[pl.ds(s0, R), :], y_v, s0_)
        c2 = pltpu.make_async_copy(k_hbm.at[pl.ds(s0, R), :], k_v, s1_)
        c3 = pltpu.make_async_copy(w_hbm.at[pl.ds(s0, R), :], w_v, s2_)
        c1.start(); c2.start(); c3.start()
        c1.wait(); c2.wait(); c3.wait()
        yw = y_v[:, :ci] * w_v[:]
        kcol = k_v[:]
        acc = jnp.zeros((R, co), jnp.float32)
        for j in range(KB):
            kk = g * KB + j
            m = (kcol == kk).astype(jnp.float32)
            acc = acc + jnp.dot(yw * m, W_ref[j],
                                preferred_element_type=jnp.float32)
        z_v[:] = jnp.concatenate(
            [acc, jnp.zeros((R, DPAD - co), jnp.float32)], axis=1)
        co_ = pltpu.make_async_copy(z_v, z_hbm.at[pl.ds(s0, R), :], s3_)
        co_.start(); co_.wait()
        return carry

    lax.fori_loop(0, nch, body, 0)


def _grouped_matmul(Y, keys2, wv2, Wp, off, ci, co, nalloc):
    kern = functools.partial(_grouped_kernel, ci, co)
    grid_spec = pltpu.PrefetchScalarGridSpec(
        num_scalar_prefetch=1,
        grid=(G,),
        in_specs=[
            pl.BlockSpec(memory_space=pl.ANY),
            pl.BlockSpec(memory_space=pl.ANY),
            pl.BlockSpec(memory_space=pl.ANY),
            pl.BlockSpec((KB, ci, co), lambda g, off: (g, 0, 0)),
        ],
        out_specs=pl.BlockSpec(memory_space=pl.ANY),
        scratch_shapes=[
            pltpu.VMEM((R, DPAD), jnp.float32),
            pltpu.VMEM((R, 1), jnp.int32),
            pltpu.VMEM((R, 1), jnp.float32),
            pltpu.VMEM((R, DPAD), jnp.float32),
            pltpu.SemaphoreType.DMA,
            pltpu.SemaphoreType.DMA,
            pltpu.SemaphoreType.DMA,
            pltpu.SemaphoreType.DMA,
        ],
    )
    return pl.pallas_call(
        kern,
        grid_spec=grid_spec,
        out_shape=jax.ShapeDtypeStruct((nalloc, DPAD), jnp.float32),
    )(off, Y, keys2, wv2, Wp)


def _epilogue_kernel(ci, co, out_w, elu, agg_ref, h_ref, root_ref, b_ref,
                     s_ref, t_ref, o_ref):
    v = (agg_ref[:, :co]
         + jnp.dot(h_ref[:, :ci], root_ref[:],
                   preferred_element_type=jnp.float32)
         + b_ref[:])
    v = v * s_ref[:] + t_ref[:]
    if elu:
        v = jnp.where(v > 0, v, jnp.exp(v) - 1.0)
    if out_w > co:
        v = jnp.concatenate([v, jnp.zeros((v.shape[0], out_w - co),
                                          jnp.float32)], axis=1)
    o_ref[:] = v


def _epilogue(agg, h_pad, root, bias, bn_s, bn_t, elu, out_w):
    ci, co = root.shape
    NB = 400
    kern = functools.partial(_epilogue_kernel, ci, co, out_w, elu)
    return pl.pallas_call(
        kern,
        grid=(N_NODES // NB,),
        in_specs=[
            pl.BlockSpec((NB, DPAD), lambda i: (i, 0)),
            pl.BlockSpec((NB, DPAD), lambda i: (i, 0)),
            pl.BlockSpec((ci, co), lambda i: (0, 0)),
            pl.BlockSpec((1, co), lambda i: (0, 0)),
            pl.BlockSpec((1, co), lambda i: (0, 0)),
            pl.BlockSpec((1, co), lambda i: (0, 0)),
        ],
        out_specs=pl.BlockSpec((NB, out_w), lambda i: (i, 0)),
        out_shape=jax.ShapeDtypeStruct((N_NODES, out_w), jnp.float32),
    )(agg, h_pad, root, bias.reshape(1, co), bn_s.reshape(1, co),
      bn_t.reshape(1, co))


def kernel(x, edge_index, edge_attr, W1, root1, b1, g1, be1, rm1, rv1,
           W2, root2, b2, g2, be2, rm2, rv2, W3, root3, b3, g3, be3, rm3,
           rv3, W4, root4, b4):
    E = edge_index.shape[1]
    P = E * S
    NROWS = ((P + 8191) // 8192 + 2) * 8192
    ALLOC = NROWS + R

    src = edge_index[0]
    dst = edge_index[1]
    w8, k8 = _basis_kidx(edge_attr)

    key = k8.reshape(-1)
    wv = w8.reshape(-1)
    srcp = jnp.broadcast_to(src[:, None], (E, S)).reshape(-1)
    dstp = jnp.broadcast_to(dst[:, None], (E, S)).reshape(-1)

    pad = ALLOC - P
    key = jnp.concatenate([key, jnp.full((pad,), GKEYS - 1, jnp.int32)])
    wv = jnp.concatenate([wv, jnp.zeros((pad,), jnp.float32)])
    srcp = jnp.concatenate([srcp, jnp.zeros((pad,), jnp.int32)])
    dstp = jnp.concatenate([dstp, jnp.full((pad,), NSEG - 1, jnp.int32)])

    order = jnp.argsort(key)
    key_s = key[order]
    wv_s = wv[order]
    src_s = srcp[order]
    dst_s = dstp[order]

    hist = jnp.bincount(key_s[:NROWS], length=GKEYS)
    off = jnp.concatenate([jnp.zeros((1,), jnp.int32),
                           jnp.cumsum(hist).astype(jnp.int32)])

    keys2 = key_s.reshape(ALLOC, 1)
    wv2 = wv_s.reshape(ALLOC, 1)

    dims = [(50, 75), (75, 100), (100, 75), (75, 50)]
    Ws = [W1, W2, W3, W4]
    roots = [root1, root2, root3, root4]
    biases = [b1, b2, b3, b4]
    bns = [(g1, be1, rm1, rv1), (g2, be2, rm2, rv2), (g3, be3, rm3, rv3),
           None]

    h_pad = jnp.concatenate(
        [x, jnp.zeros((N_NODES, DPAD - x.shape[1]), jnp.float32)], axis=1)

    out = None
    for li in range(4):
        ci, co = dims[li]
        Wp = jnp.concatenate(
            [Ws[li], jnp.zeros((GKEYS - KC, ci, co), jnp.float32)], axis=0)
        Y = jnp.take(h_pad, src_s, axis=0)
        Z = _grouped_matmul(Y, keys2, wv2, Wp, off, ci, co, ALLOC)
        agg = jax.ops.segment_sum(Z, dst_s, num_segments=NSEG)[:N_NODES]
        if bns[li] is not None:
            g_, be_, rm_, rv_ = bns[li]
            bn_s = g_ / jnp.sqrt(rv_ + EPS)
            bn_t = be_ - rm_ * bn_s
            elu = True
            out_w = DPAD
        else:
            bn_s = jnp.ones((co,), jnp.float32)
            bn_t = jnp.zeros((co,), jnp.float32)
            elu = False
            out_w = co
        res = _epilogue(agg, h_pad, roots[li], biases[li], bn_s, bn_t,
                        elu, out_w)
        if li < 3:
            h_pad = res
        else:
            out = res
    return out


# speedup vs baseline: 1.6630x; 1.2198x over previous
import functools
import jax
import jax.numpy as jnp
from jax import lax
from jax.experimental import pallas as pl
from jax.experimental.pallas import tpu as pltpu

KS = 25
DIM = 3
S = 8
KC = KS ** DIM
KB = 8
GKEYS = 15632
G = GKEYS // KB
R = 512
DPAD = 128
EPS = 1e-5
N_NODES = 10000
NSEG = 10240


def _basis_kidx(edge_attr):
    u = jnp.clip(edge_attr, 0.0, 1.0)
    v = u * (KS - 1)
    lo = jnp.floor(v)
    frac = v - lo
    lo_i = lo.astype(jnp.int32)
    bits = jnp.array([[(s >> d) & 1 for d in range(DIM)] for s in range(S)],
                     jnp.int32)
    fd = frac[:, None, :]
    w = jnp.prod(jnp.where(bits[None] == 1, fd, 1.0 - fd), axis=-1)
    strides = jnp.array([1, KS, KS * KS], jnp.int32)
    idx = jnp.sum(jnp.clip(lo_i[:, None, :] + bits[None], 0, KS - 1)
                  * strides, axis=-1)
    return w, idx.astype(jnp.int32)


def _grouped_kernel(ci, co, off_ref, y_hbm, kw_hbm, W_ref, z_hbm,
                    y_v, kw_v, z_v, s0_, s1_, s3_):
    g = pl.program_id(0)
    start = off_ref[g * KB]
    end = off_ref[(g + 1) * KB]
    nch = (end - start + (R - 1)) // R

    def body(i, carry):
        s0 = start + i * R
        c1 = pltpu.make_async_copy(y_hbm.at[pl.ds(s0, R), :], y_v, s0_)
        c2 = pltpu.make_async_copy(kw_hbm.at[pl.ds(s0, R), :], kw_v, s1_)
        c1.start(); c2.start()
        c1.wait(); c2.wait()
        yw = y_v[:, :ci] * kw_v[:, 1:2]
        kcol = kw_v[:, 0:1]
        acc = jnp.zeros((R, co), jnp.float32)
        for j in range(KB):
            kk = g * KB + j
            m = (kcol == lax.convert_element_type(kk, jnp.float32)).astype(jnp.float32)
            acc = acc + jnp.dot(yw * m, W_ref[j],
                                preferred_element_type=jnp.float32)
        z_v[:] = jnp.concatenate(
            [acc, jnp.zeros((R, DPAD - co), jnp.float32)], axis=1)
        co_ = pltpu.make_async_copy(z_v, z_hbm.at[pl.ds(s0, R), :], s3_)
        co_.start(); co_.wait()
        return carry

    lax.fori_loop(0, nch, body, 0)


def _grouped_matmul(Y, keys2, Wp, off, ci, co, nalloc):
    kern = functools.partial(_grouped_kernel, ci, co)
    grid_spec = pltpu.PrefetchScalarGridSpec(
        num_scalar_prefetch=1,
        grid=(G,),
        in_specs=[
            pl.BlockSpec(memory_space=pl.ANY),
            pl.BlockSpec(memory_space=pl.ANY),
            pl.BlockSpec((KB, ci, co), lambda g, off: (g, 0, 0)),
        ],
        out_specs=pl.BlockSpec(memory_space=pl.ANY),
        scratch_shapes=[
            pltpu.VMEM((R, DPAD), jnp.float32),
            pltpu.VMEM((R, 2), jnp.float32),
            pltpu.VMEM((R, DPAD), jnp.float32),
            pltpu.SemaphoreType.DMA,
            pltpu.SemaphoreType.DMA,
            pltpu.SemaphoreType.DMA,
        ],
    )
    return pl.pallas_call(
        kern,
        grid_spec=grid_spec,
        out_shape=jax.ShapeDtypeStruct((nalloc, DPAD), jnp.float32),
    )(off, Y, keys2, Wp)


def _epilogue_kernel(ci, co, out_w, elu, agg_ref, h_ref, root_ref, b_ref,
                     s_ref, t_ref, o_ref):
    v = (agg_ref[:, :co]
         + jnp.dot(h_ref[:, :ci], root_ref[:],
                   preferred_element_type=jnp.float32)
         + b_ref[:])
    v = v * s_ref[:] + t_ref[:]
    if elu:
        v = jnp.where(v > 0, v, jnp.exp(v) - 1.0)
    if out_w > co:
        v = jnp.concatenate([v, jnp.zeros((v.shape[0], out_w - co),
                                          jnp.float32)], axis=1)
    o_ref[:] = v


def _epilogue(agg, h_pad, root, bias, bn_s, bn_t, elu, out_w):
    ci, co = root.shape
    NB = 400
    kern = functools.partial(_epilogue_kernel, ci, co, out_w, elu)
    return pl.pallas_call(
        kern,
        grid=(N_NODES // NB,),
        in_specs=[
            pl.BlockSpec((NB, DPAD), lambda i: (i, 0)),
            pl.BlockSpec((NB, DPAD), lambda i: (i, 0)),
            pl.BlockSpec((ci, co), lambda i: (0, 0)),
            pl.BlockSpec((1, co), lambda i: (0, 0)),
            pl.BlockSpec((1, co), lambda i: (0, 0)),
            pl.BlockSpec((1, co), lambda i: (0, 0)),
        ],
        out_specs=pl.BlockSpec((NB, out_w), lambda i: (i, 0)),
        out_shape=jax.ShapeDtypeStruct((N_NODES, out_w), jnp.float32),
    )(agg, h_pad, root, bias.reshape(1, co), bn_s.reshape(1, co),
      bn_t.reshape(1, co))


def kernel(x, edge_index, edge_attr, W1, root1, b1, g1, be1, rm1, rv1,
           W2, root2, b2, g2, be2, rm2, rv2, W3, root3, b3, g3, be3, rm3,
           rv3, W4, root4, b4):
    E = edge_index.shape[1]
    P = E * S
    NROWS = ((P + 8191) // 8192 + 2) * 8192
    ALLOC = NROWS + R

    src = edge_index[0]
    dst = edge_index[1]
    w8, k8 = _basis_kidx(edge_attr)

    key = k8.reshape(-1)
    wv = w8.reshape(-1)
    srcp = jnp.broadcast_to(src[:, None], (E, S)).reshape(-1)
    dstp = jnp.broadcast_to(dst[:, None], (E, S)).reshape(-1)

    pad = ALLOC - P
    key = jnp.concatenate([key, jnp.full((pad,), GKEYS - 1, jnp.int32)])
    wv = jnp.concatenate([wv, jnp.zeros((pad,), jnp.float32)])
    srcp = jnp.concatenate([srcp, jnp.zeros((pad,), jnp.int32)])
    dstp = jnp.concatenate([dstp, jnp.full((pad,), NSEG - 1, jnp.int32)])

    order = jnp.argsort(key)
    key_s = key[order]
    wv_s = wv[order]
    src_s = srcp[order]
    dst_s = dstp[order]

    hist = jnp.bincount(key_s[:NROWS], length=GKEYS)
    off = jnp.concatenate([jnp.zeros((1,), jnp.int32),
                           jnp.cumsum(hist).astype(jnp.int32)])

    keys2 = jnp.stack([key_s.astype(jnp.float32), wv_s], axis=1)

    dims = [(50, 75), (75, 100), (100, 75), (75, 50)]
    Ws = [W1, W2, W3, W4]
    roots = [root1, root2, root3, root4]
    biases = [b1, b2, b3, b4]
    bns = [(g1, be1, rm1, rv1), (g2, be2, rm2, rv2), (g3, be3, rm3, rv3),
           None]

    h_pad = jnp.concatenate(
        [x, jnp.zeros((N_NODES, DPAD - x.shape[1]), jnp.float32)], axis=1)

    out = None
    for li in range(4):
        ci, co = dims[li]
        Wp = jnp.concatenate(
            [Ws[li], jnp.zeros((GKEYS - KC, ci, co), jnp.float32)], axis=0)
        Y = jnp.take(h_pad, src_s, axis=0)
        Z = _grouped_matmul(Y, keys2, Wp, off, ci, co, ALLOC)
        agg = jax.ops.segment_sum(Z, dst_s, num_segments=NSEG)[:N_NODES]
        if bns[li] is not None:
            g_, be_, rm_, rv_ = bns[li]
            bn_s = g_ / jnp.sqrt(rv_ + EPS)
            bn_t = be_ - rm_ * bn_s
            elu = True
            out_w = DPAD
        else:
            bn_s = jnp.ones((co,), jnp.float32)
            bn_t = jnp.zeros((co,), jnp.float32)
            elu = False
            out_w = co
        res = _epilogue(agg, h_pad, roots[li], biases[li], bn_s, bn_t,
                        elu, out_w)
        if li < 3:
            h_pad = res
        else:
            out = res
    return out
